# Initial kernel scaffold; baseline (speedup 1.0000x reference)
import jax
import jax.numpy as jnp
from jax.experimental import pallas as pl


def kernel(x, emb, Wq, Wk, v_w, fc_w, fc_b, w1, b1, w2, b2):
    raise NotImplementedError("write your pallas kernel here")


# speedup vs baseline: 8.3791x; 8.3791x over previous
import functools

import jax
import jax.numpy as jnp
from jax.experimental import pallas as pl
from jax.experimental.pallas import tpu as pltpu

_N = 10000
_W = 128
_D = 64
_H = 64
_K = 16
_B = 4

_NEG = -3.4e38


def _topk_body(n_real, k, blk, emb_ref, wq_ref, wk_ref, idx_ref, a_ref, bm_ref,
               e_scr):
    pid = pl.program_id(0)

    @pl.when(pid == 0)
    def _():
        e = emb_ref[...]
        nrm = jnp.sqrt(jnp.sum(e * e, axis=1, keepdims=True)) + 1e-8
        e_scr[...] = e / nrm

    rows_raw = emb_ref[pl.ds(pid * blk, blk), :]
    a_ref[...] = jax.lax.dot_general(
        rows_raw, wq_ref[...], (((1,), (1,)), ((), ())),
        preferred_element_type=jnp.float32)
    bm_ref[...] = jax.lax.dot_general(
        rows_raw, wk_ref[...], (((1,), (1,)), ((), ())),
        preferred_element_type=jnp.float32)

    e_all = e_scr[...]
    rows = e_scr[pl.ds(pid * blk, blk), :]
    sim = jax.lax.dot_general(
        rows, e_all, (((1,), (1,)), ((), ())),
        preferred_element_type=jnp.float32)

    np_cols = e_all.shape[0]
    col = jax.lax.broadcasted_iota(jnp.int32, (blk, np_cols), 1)
    rowid = pid * blk + jax.lax.broadcasted_iota(jnp.int32, (blk, np_cols), 0)
    sim = jnp.where(col == rowid, sim - 1e9, sim)
    sim = jnp.where(col >= n_real, _NEG, sim)

    outs = []
    for _ in range(k):
        m = jnp.max(sim, axis=1, keepdims=True)
        idx_t = jnp.min(jnp.where(sim >= m, col, jnp.int32(2 ** 30)), axis=1)
        outs.append(idx_t)
        sim = jnp.where(col == idx_t[:, None], _NEG, sim)
    idx_ref[...] = jnp.stack(outs, axis=1)


def _run_topk(emb_p, wq, wk, n_real, k, blk, interpret=False):
    np_, d = emb_p.shape
    grid = (np_ // blk,)
    return pl.pallas_call(
        functools.partial(_topk_body, n_real, k, blk),
        grid=grid,
        in_specs=[
            pl.BlockSpec((np_, d), lambda i: (0, 0)),
            pl.BlockSpec((d, d), lambda i: (0, 0)),
            pl.BlockSpec((d, d), lambda i: (0, 0)),
        ],
        out_specs=[
            pl.BlockSpec((blk, k), lambda i: (i, 0)),
            pl.BlockSpec((blk, d), lambda i: (i, 0)),
            pl.BlockSpec((blk, d), lambda i: (i, 0)),
        ],
        out_shape=[
            jax.ShapeDtypeStruct((np_, k), jnp.int32),
            jax.ShapeDtypeStruct((np_, d), jnp.float32),
            jax.ShapeDtypeStruct((np_, d), jnp.float32),
        ],
        scratch_shapes=[pltpu.VMEM((np_, d), jnp.float32)],
        interpret=interpret,
    )(emb_p, wq, wk)


def _edge_body(k, blk, chunk, idx_ref, a_ref, bm_ref, vt_ref, xt_ref, agg_ref):
    np_cols = a_ref.shape[0]
    d = a_ref.shape[1]
    bw = xt_ref.shape[1]
    idxb = idx_ref[...]
    bmb = bm_ref[...]
    vt = vt_ref[...]

    scores = []
    for t in range(k):
        tgt = idxb[:, t][:, None]
        ak = jnp.zeros((blk, d), dtype=jnp.float32)
        for c in range(0, np_cols, chunk):
            cio = c + jax.lax.broadcasted_iota(jnp.int32, (blk, chunk), 1)
            oh = (cio == tgt).astype(jnp.float32)
            ak = ak + jnp.dot(oh, a_ref[pl.ds(c, chunk), :],
                              preferred_element_type=jnp.float32)
        z = jnp.tanh(ak + bmb)
        scores.append(jnp.dot(z, vt, preferred_element_type=jnp.float32)[:, 0])
    s = jnp.stack(scores, axis=1)

    mx = jnp.max(s, axis=1, keepdims=True)
    e = jnp.exp(s - mx)
    attn = e / (jnp.sum(e, axis=1, keepdims=True) + 1e-8)

    acc = jnp.zeros((blk, bw), dtype=jnp.float32)
    for c in range(0, np_cols, chunk):
        cio = c + jax.lax.broadcasted_iota(jnp.int32, (blk, chunk), 1)
        p = jnp.zeros((blk, chunk), dtype=jnp.float32)
        for t in range(k):
            p = p + jnp.where(cio == idxb[:, t][:, None],
                              attn[:, t][:, None], 0.0)
        acc = acc + jnp.dot(p, xt_ref[pl.ds(c, chunk), :],
                            preferred_element_type=jnp.float32)
    agg_ref[...] = acc


def _run_edge_tc(idx, a, bm, vt, xt, k, blk, chunk, interpret=False):
    np_, d = a.shape
    bw = xt.shape[1]
    grid = (np_ // blk,)
    return pl.pallas_call(
        functools.partial(_edge_body, k, blk, chunk),
        grid=grid,
        in_specs=[
            pl.BlockSpec((blk, k), lambda i: (i, 0)),
            pl.BlockSpec((np_, d), lambda i: (0, 0)),
            pl.BlockSpec((blk, d), lambda i: (i, 0)),
            pl.BlockSpec((d, 1), lambda i: (0, 0)),
            pl.BlockSpec((np_, bw), lambda i: (0, 0)),
        ],
        out_specs=pl.BlockSpec((blk, bw), lambda i: (i, 0)),
        out_shape=jax.ShapeDtypeStruct((np_, bw), jnp.float32),
        interpret=interpret,
    )(idx, a, bm, vt, xt)


def _mlp_body(x_ref, agg_ref, fwx_ref, fwa_ref, fb_ref, w1t_ref, b1_ref,
              w2t_ref, b2_ref, out_ref):
    h = jnp.dot(x_ref[...], fwx_ref[...], preferred_element_type=jnp.float32)
    h = h + jnp.dot(agg_ref[...], fwa_ref[...],
                    preferred_element_type=jnp.float32)
    h = jax.nn.relu(h + fb_ref[...])
    h1 = jax.nn.relu(jnp.dot(h, w1t_ref[...],
                             preferred_element_type=jnp.float32) + b1_ref[...])
    out_ref[...] = jnp.dot(h1, w2t_ref[...],
                           preferred_element_type=jnp.float32) + b2_ref[...]


def _run_mlp(xf, aggf, fwx, fwa, fb, w1t, b1, w2t, b2, blk, interpret=False):
    rows, w = xf.shape
    h = fwx.shape[1]
    h2 = w1t.shape[1]
    grid = (rows // blk,)
    return pl.pallas_call(
        _mlp_body,
        grid=grid,
        in_specs=[
            pl.BlockSpec((blk, w), lambda i: (i, 0)),
            pl.BlockSpec((blk, w), lambda i: (i, 0)),
            pl.BlockSpec((w, h), lambda i: (0, 0)),
            pl.BlockSpec((w, h), lambda i: (0, 0)),
            pl.BlockSpec((1, h), lambda i: (0, 0)),
            pl.BlockSpec((h, h2), lambda i: (0, 0)),
            pl.BlockSpec((1, h2), lambda i: (0, 0)),
            pl.BlockSpec((h2, 1), lambda i: (0, 0)),
            pl.BlockSpec((1, 1), lambda i: (0, 0)),
        ],
        out_specs=pl.BlockSpec((blk, 1), lambda i: (i, 0)),
        out_shape=jax.ShapeDtypeStruct((rows, 1), jnp.float32),
        interpret=interpret,
    )(xf, aggf, fwx, fwa, fb, w1t, b1, w2t, b2)


def _pipeline(x, emb, Wq, Wk, v_w, fc_w, fc_b, w1, b1, w2, b2,
              n, w, d, k, b, blk_topk, blk_edge, chunk, blk_mlp,
              interpret=False):
    np_ = ((n + blk_topk - 1) // blk_topk) * blk_topk
    emb_p = jnp.pad(emb, ((0, np_ - n), (0, 0)))

    idx, a_proj, bm_proj = _run_topk(emb_p, Wq, Wk, n, k, blk_topk,
                                     interpret=interpret)

    xt = jnp.pad(x.transpose(1, 0, 2).reshape(n, b * w), ((0, np_ - n), (0, 0)))
    vt = v_w.reshape(d, 1)

    aggt = _run_edge_tc(idx, a_proj, bm_proj, vt, xt, k, blk_edge, chunk,
                        interpret=interpret)

    xf = xt.reshape(np_ * b, w)
    aggf = aggt.reshape(np_ * b, w)
    fwx = fc_w[:, :w].T
    fwa = fc_w[:, w:].T
    w1t = w1.T
    w2t = w2.T
    pred = _run_mlp(xf, aggf, fwx, fwa, fc_b[None, :], w1t, b1[None, :], w2t,
                    b2[None, :], blk_mlp, interpret=interpret)
    return pred.reshape(np_, b)[:n, :].T


def kernel(x, emb, Wq, Wk, v_w, fc_w, fc_b, w1, b1, w2, b2):
    return _pipeline(x, emb, Wq, Wk, v_w, fc_w, fc_b, w1, b1, w2, b2,
                     n=_N, w=_W, d=_D, k=_K, b=_B,
                     blk_topk=256, blk_edge=128, chunk=2048, blk_mlp=512)


# speedup vs baseline: 10.3212x; 1.2318x over previous
import functools

import jax
import jax.numpy as jnp
from jax import lax
from jax.experimental import pallas as pl
from jax.experimental.pallas import tpu as pltpu
from jax.experimental.pallas import tpu_sc as plsc

_N = 10000
_W = 128
_D = 64
_H = 64
_K = 16
_B = 4

_NEG = -3.4e38


def _topk_body(n_real, k, blk, emb_ref, wq_ref, wk_ref, idx_ref, a_ref, bm_ref,
               e_scr):
    pid = pl.program_id(0)

    @pl.when(pid == 0)
    def _():
        e = emb_ref[...]
        nrm = jnp.sqrt(jnp.sum(e * e, axis=1, keepdims=True)) + 1e-8
        e_scr[...] = e / nrm

    rows_raw = emb_ref[pl.ds(pid * blk, blk), :]
    aq = jax.lax.dot_general(
        rows_raw, wq_ref[...], (((1,), (1,)), ((), ())),
        preferred_element_type=jnp.float32)
    a_ref[...] = jnp.concatenate([aq, jnp.zeros_like(aq)], axis=1)
    bm_ref[...] = jax.lax.dot_general(
        rows_raw, wk_ref[...], (((1,), (1,)), ((), ())),
        preferred_element_type=jnp.float32)

    e_all = e_scr[...]
    rows = e_scr[pl.ds(pid * blk, blk), :]
    sim = jax.lax.dot_general(
        rows, e_all, (((1,), (1,)), ((), ())),
        preferred_element_type=jnp.float32)

    np_cols = e_all.shape[0]
    col = jax.lax.broadcasted_iota(jnp.int32, (blk, np_cols), 1)
    rowid = pid * blk + jax.lax.broadcasted_iota(jnp.int32, (blk, np_cols), 0)
    sim = jnp.where(col == rowid, sim - 1e9, sim)
    sim = jnp.where(col >= n_real, _NEG, sim)

    outs = []
    for _ in range(k):
        m = jnp.max(sim, axis=1, keepdims=True)
        idx_t = jnp.min(jnp.where(sim >= m, col, jnp.int32(2 ** 30)), axis=1)
        outs.append(idx_t)
        sim = jnp.where(col == idx_t[:, None], _NEG, sim)
    idx_ref[...] = jnp.stack(outs, axis=1)


def _run_topk(emb_p, wq, wk, n_real, k, blk, interpret=False):
    np_, d = emb_p.shape
    grid = (np_ // blk,)
    return pl.pallas_call(
        functools.partial(_topk_body, n_real, k, blk),
        grid=grid,
        in_specs=[
            pl.BlockSpec((np_, d), lambda i: (0, 0)),
            pl.BlockSpec((d, d), lambda i: (0, 0)),
            pl.BlockSpec((d, d), lambda i: (0, 0)),
        ],
        out_specs=[
            pl.BlockSpec((blk, k), lambda i: (i, 0)),
            pl.BlockSpec((blk, 2 * d), lambda i: (i, 0)),
            pl.BlockSpec((blk, d), lambda i: (i, 0)),
        ],
        out_shape=[
            jax.ShapeDtypeStruct((np_, k), jnp.int32),
            jax.ShapeDtypeStruct((np_, 2 * d), jnp.float32),
            jax.ShapeDtypeStruct((np_, d), jnp.float32),
        ],
        scratch_shapes=[pltpu.VMEM((np_, d), jnp.float32)],
        interpret=interpret,
    )(emb_p, wq, wk)


def _edge_body(k, blk, chunk, idx_ref, a_ref, bm_ref, vt_ref, xt_ref, agg_ref):
    np_cols = a_ref.shape[0]
    d = a_ref.shape[1]
    bw = xt_ref.shape[1]
    idxb = idx_ref[...]
    bmb = bm_ref[...]
    vt = vt_ref[...]

    scores = []
    for t in range(k):
        tgt = idxb[:, t][:, None]
        ak = jnp.zeros((blk, d), dtype=jnp.float32)
        for c in range(0, np_cols, chunk):
            cio = c + jax.lax.broadcasted_iota(jnp.int32, (blk, chunk), 1)
            oh = (cio == tgt).astype(jnp.float32)
            ak = ak + jnp.dot(oh, a_ref[pl.ds(c, chunk), :],
                              preferred_element_type=jnp.float32)
        z = jnp.tanh(ak + bmb)
        scores.append(jnp.dot(z, vt, preferred_element_type=jnp.float32)[:, 0])
    s = jnp.stack(scores, axis=1)

    mx = jnp.max(s, axis=1, keepdims=True)
    e = jnp.exp(s - mx)
    attn = e / (jnp.sum(e, axis=1, keepdims=True) + 1e-8)

    acc = jnp.zeros((blk, bw), dtype=jnp.float32)
    for c in range(0, np_cols, chunk):
        cio = c + jax.lax.broadcasted_iota(jnp.int32, (blk, chunk), 1)
        p = jnp.zeros((blk, chunk), dtype=jnp.float32)
        for t in range(k):
            p = p + jnp.where(cio == idxb[:, t][:, None],
                              attn[:, t][:, None], 0.0)
        acc = acc + jnp.dot(p, xt_ref[pl.ds(c, chunk), :],
                            preferred_element_type=jnp.float32)
    agg_ref[...] = acc


def _run_edge_tc(idx, a, bm, vt, xt, k, blk, chunk, interpret=False):
    np_, d = a.shape
    bw = xt.shape[1]
    grid = (np_ // blk,)
    return pl.pallas_call(
        functools.partial(_edge_body, k, blk, chunk),
        grid=grid,
        in_specs=[
            pl.BlockSpec((blk, k), lambda i: (i, 0)),
            pl.BlockSpec((np_, d), lambda i: (0, 0)),
            pl.BlockSpec((blk, d), lambda i: (i, 0)),
            pl.BlockSpec((d, 1), lambda i: (0, 0)),
            pl.BlockSpec((np_, bw), lambda i: (0, 0)),
        ],
        out_specs=pl.BlockSpec((blk, bw), lambda i: (i, 0)),
        out_shape=jax.ShapeDtypeStruct((np_, bw), jnp.float32),
        interpret=interpret,
    )(idx, a, bm, vt, xt)


_SC_NC = 2
_SC_NS = 16
_SC_L = 16


def _tanh_sc(z):
    ez = jnp.exp(z + z)
    return 1.0 - 2.0 / (ez + 1.0)


def _make_edge_sc(np_, k, d, bw, nb):
    nw = _SC_NC * _SC_NS
    npw = np_ // nw
    groups = npw // nb
    cpr = bw // _SC_L
    dch = d // _SC_L
    mesh = plsc.VectorSubcoreMesh(core_axis_name="c", subcore_axis_name="s")

    @functools.partial(
        pl.kernel, mesh=mesh,
        compiler_params=pltpu.CompilerParams(needs_layout_passes=False),
        out_type=jax.ShapeDtypeStruct((np_, bw), jnp.float32),
        scratch_types=[
            pltpu.VMEM((nb * k,), jnp.int32),
            pltpu.VMEM((nb * k, 2 * d), jnp.float32),
            pltpu.VMEM((nb * k, bw), jnp.float32),
            pltpu.VMEM((nb, d), jnp.float32),
            pltpu.VMEM((nb, bw), jnp.float32),
            pltpu.VMEM((d,), jnp.float32),
            pltpu.SemaphoreType.DMA,
        ],
    )
    def edge_sc(idxf_hbm, a_hbm, bm_hbm, vt_hbm, xt_hbm, out_hbm,
                idx_v, arows_v, xrows_v, bm_v, agg_v, vt_v, sem):
        wid = lax.axis_index("s") * _SC_NC + lax.axis_index("c")
        base = wid * npw
        pltpu.sync_copy(vt_hbm, vt_v)
        lanes = lax.iota(jnp.int32, _SC_L)

        def group_body(g, carry):
            node0 = base + g * nb
            pltpu.sync_copy(idxf_hbm.at[pl.ds(node0 * k, nb * k)], idx_v)
            pltpu.async_copy(a_hbm.at[idx_v], arows_v, sem).wait()
            pltpu.async_copy(xt_hbm.at[idx_v], xrows_v, sem).wait()
            pltpu.sync_copy(bm_hbm.at[pl.ds(node0, nb)], bm_v)

            def node_body(nn, ncarry):
                row0 = nn * k
                bmc = [bm_v[nn, pl.ds(c * _SC_L, _SC_L)] for c in range(dch)]
                vc = [vt_v[pl.ds(c * _SC_L, _SC_L)] for c in range(dch)]
                score = jnp.zeros((_SC_L,), jnp.float32)
                for kk in range(k):
                    part = jnp.zeros((_SC_L,), jnp.float32)
                    for c in range(dch):
                        z = arows_v[row0 + kk,
                                    pl.ds(c * _SC_L, _SC_L)] + bmc[c]
                        part = part + vc[c] * _tanh_sc(z)
                    score = jnp.where(lanes == kk, jnp.sum(part), score)
                m = jnp.max(score)
                e = jnp.exp(score - m)
                attn = e / (jnp.sum(e) + 1e-8)

                accs = [jnp.zeros((_SC_L,), jnp.float32) for _ in range(cpr)]
                for kk in range(k):
                    ak = jnp.sum(jnp.where(lanes == kk, attn, 0.0))
                    for c in range(cpr):
                        accs[c] = accs[c] + ak * xrows_v[
                            row0 + kk, pl.ds(c * _SC_L, _SC_L)]
                for c in range(cpr):
                    agg_v[nn, pl.ds(c * _SC_L, _SC_L)] = accs[c]
                return ncarry

            lax.fori_loop(0, nb, node_body, 0)
            pltpu.sync_copy(agg_v, out_hbm.at[pl.ds(node0, nb)])
            return carry

        lax.fori_loop(0, groups, group_body, 0)

    return edge_sc


def _mlp_body(x_ref, agg_ref, fwx_ref, fwa_ref, fb_ref, w1t_ref, b1_ref,
              w2t_ref, b2_ref, out_ref):
    h = jnp.dot(x_ref[...], fwx_ref[...], preferred_element_type=jnp.float32)
    h = h + jnp.dot(agg_ref[...], fwa_ref[...],
                    preferred_element_type=jnp.float32)
    h = jax.nn.relu(h + fb_ref[...])
    h1 = jax.nn.relu(jnp.dot(h, w1t_ref[...],
                             preferred_element_type=jnp.float32) + b1_ref[...])
    out_ref[...] = jnp.dot(h1, w2t_ref[...],
                           preferred_element_type=jnp.float32) + b2_ref[...]


def _run_mlp(xf, aggf, fwx, fwa, fb, w1t, b1, w2t, b2, blk, interpret=False):
    rows, w = xf.shape
    h = fwx.shape[1]
    h2 = w1t.shape[1]
    grid = (rows // blk,)
    return pl.pallas_call(
        _mlp_body,
        grid=grid,
        in_specs=[
            pl.BlockSpec((blk, w), lambda i: (i, 0)),
            pl.BlockSpec((blk, w), lambda i: (i, 0)),
            pl.BlockSpec((w, h), lambda i: (0, 0)),
            pl.BlockSpec((w, h), lambda i: (0, 0)),
            pl.BlockSpec((1, h), lambda i: (0, 0)),
            pl.BlockSpec((h, h2), lambda i: (0, 0)),
            pl.BlockSpec((1, h2), lambda i: (0, 0)),
            pl.BlockSpec((h2, 1), lambda i: (0, 0)),
            pl.BlockSpec((1, 1), lambda i: (0, 0)),
        ],
        out_specs=pl.BlockSpec((blk, 1), lambda i: (i, 0)),
        out_shape=jax.ShapeDtypeStruct((rows, 1), jnp.float32),
        interpret=interpret,
    )(xf, aggf, fwx, fwa, fb, w1t, b1, w2t, b2)


def _pipeline(x, emb, Wq, Wk, v_w, fc_w, fc_b, w1, b1, w2, b2,
              n, w, d, k, b, blk_topk, blk_edge, chunk, blk_mlp,
              interpret=False):
    np_ = ((n + blk_topk - 1) // blk_topk) * blk_topk
    emb_p = jnp.pad(emb, ((0, np_ - n), (0, 0)))

    idx, a_proj, bm_proj = _run_topk(emb_p, Wq, Wk, n, k, blk_topk,
                                     interpret=interpret)

    xt = jnp.pad(x.transpose(1, 0, 2).reshape(n, b * w), ((0, np_ - n), (0, 0)))
    vt = v_w.reshape(d, 1)

    if interpret:
        aggt = _run_edge_tc(idx, a_proj, bm_proj, vt, xt, k, blk_edge, chunk,
                            interpret=True)
    else:
        edge_sc = _make_edge_sc(np_, k, d, b * w, nb=8)
        aggt = edge_sc(idx.reshape(np_ * k), a_proj, bm_proj,
                       v_w.reshape(d), xt)

    xf = xt.reshape(np_ * b, w)
    aggf = aggt.reshape(np_ * b, w)
    fwx = fc_w[:, :w].T
    fwa = fc_w[:, w:].T
    w1t = w1.T
    w2t = w2.T
    pred = _run_mlp(xf, aggf, fwx, fwa, fc_b[None, :], w1t, b1[None, :], w2t,
                    b2[None, :], blk_mlp, interpret=interpret)
    return pred.reshape(np_, b)[:n, :].T


def kernel(x, emb, Wq, Wk, v_w, fc_w, fc_b, w1, b1, w2, b2):
    return _pipeline(x, emb, Wq, Wk, v_w, fc_w, fc_b, w1, b1, w2, b2,
                     n=_N, w=_W, d=_D, k=_K, b=_B,
                     blk_topk=256, blk_edge=128, chunk=2048, blk_mlp=512)


# speedup vs baseline: 10.7255x; 1.0392x over previous
import functools

import jax
import jax.numpy as jnp
from jax import lax
from jax.experimental import pallas as pl
from jax.experimental.pallas import tpu as pltpu
from jax.experimental.pallas import tpu_sc as plsc

_N = 10000
_W = 128
_D = 64
_H = 64
_K = 16
_B = 4

_NEG = -3.4e38


def _topk_body(n_real, k, blk, emb_ref, wq_ref, wk_ref, idx_ref, a_ref, bm_ref,
               e_scr):
    pid = pl.program_id(0)

    @pl.when(pid == 0)
    def _():
        e = emb_ref[...]
        nrm = jnp.sqrt(jnp.sum(e * e, axis=1, keepdims=True)) + 1e-8
        e_scr[...] = e / nrm

    rows_raw = emb_ref[pl.ds(pid * blk, blk), :]
    aq = jax.lax.dot_general(
        rows_raw, wq_ref[...], (((1,), (1,)), ((), ())),
        preferred_element_type=jnp.float32)
    a_ref[...] = jnp.concatenate([aq, jnp.zeros_like(aq)], axis=1)
    bm_ref[...] = jax.lax.dot_general(
        rows_raw, wk_ref[...], (((1,), (1,)), ((), ())),
        preferred_element_type=jnp.float32)

    e_all = e_scr[...]
    rows = e_scr[pl.ds(pid * blk, blk), :]
    sim = jax.lax.dot_general(
        rows, e_all, (((1,), (1,)), ((), ())),
        preferred_element_type=jnp.float32)

    np_cols = e_all.shape[0]
    col = jax.lax.broadcasted_iota(jnp.int32, (blk, np_cols), 1)
    rowid = pid * blk + jax.lax.broadcasted_iota(jnp.int32, (blk, np_cols), 0)
    sim = jnp.where(col == rowid, sim - 1e9, sim)
    sim = jnp.where(col >= n_real, _NEG, sim)

    outs = []
    for _ in range(k):
        idx_t = jnp.argmax(sim, axis=1).astype(jnp.int32)
        outs.append(idx_t)
        sim = jnp.where(col == idx_t[:, None], _NEG, sim)
    idx_ref[...] = jnp.stack(outs, axis=1)


def _run_topk(emb_p, wq, wk, n_real, k, blk, interpret=False):
    np_, d = emb_p.shape
    grid = (np_ // blk,)
    return pl.pallas_call(
        functools.partial(_topk_body, n_real, k, blk),
        grid=grid,
        in_specs=[
            pl.BlockSpec((np_, d), lambda i: (0, 0)),
            pl.BlockSpec((d, d), lambda i: (0, 0)),
            pl.BlockSpec((d, d), lambda i: (0, 0)),
        ],
        out_specs=[
            pl.BlockSpec((blk, k), lambda i: (i, 0)),
            pl.BlockSpec((blk, 2 * d), lambda i: (i, 0)),
            pl.BlockSpec((blk, d), lambda i: (i, 0)),
        ],
        out_shape=[
            jax.ShapeDtypeStruct((np_, k), jnp.int32),
            jax.ShapeDtypeStruct((np_, 2 * d), jnp.float32),
            jax.ShapeDtypeStruct((np_, d), jnp.float32),
        ],
        scratch_shapes=[pltpu.VMEM((np_, d), jnp.float32)],
        interpret=interpret,
    )(emb_p, wq, wk)


def _edge_body(k, blk, chunk, idx_ref, a_ref, bm_ref, vt_ref, xt_ref, agg_ref):
    np_cols = a_ref.shape[0]
    aw = a_ref.shape[1]
    d = bm_ref.shape[1]
    bw = xt_ref.shape[1]
    idxb = idx_ref[...]
    bmb = bm_ref[...]
    vt = vt_ref[...]

    scores = []
    for t in range(k):
        tgt = idxb[:, t][:, None]
        ak = jnp.zeros((blk, aw), dtype=jnp.float32)
        for c in range(0, np_cols, chunk):
            cio = c + jax.lax.broadcasted_iota(jnp.int32, (blk, chunk), 1)
            oh = (cio == tgt).astype(jnp.float32)
            ak = ak + jnp.dot(oh, a_ref[pl.ds(c, chunk), :],
                              preferred_element_type=jnp.float32)
        z = jnp.tanh(ak[:, :d] + bmb)
        scores.append(jnp.dot(z, vt, preferred_element_type=jnp.float32)[:, 0])
    s = jnp.stack(scores, axis=1)

    mx = jnp.max(s, axis=1, keepdims=True)
    e = jnp.exp(s - mx)
    attn = e / (jnp.sum(e, axis=1, keepdims=True) + 1e-8)

    acc = jnp.zeros((blk, bw), dtype=jnp.float32)
    for c in range(0, np_cols, chunk):
        cio = c + jax.lax.broadcasted_iota(jnp.int32, (blk, chunk), 1)
        p = jnp.zeros((blk, chunk), dtype=jnp.float32)
        for t in range(k):
            p = p + jnp.where(cio == idxb[:, t][:, None],
                              attn[:, t][:, None], 0.0)
        acc = acc + jnp.dot(p, xt_ref[pl.ds(c, chunk), :],
                            preferred_element_type=jnp.float32)
    agg_ref[...] = acc


def _run_edge_tc(idx, a, bm, vt, xt, k, blk, chunk, interpret=False):
    np_, aw = a.shape
    d = bm.shape[1]
    bw = xt.shape[1]
    grid = (np_ // blk,)
    return pl.pallas_call(
        functools.partial(_edge_body, k, blk, chunk),
        grid=grid,
        in_specs=[
            pl.BlockSpec((blk, k), lambda i: (i, 0)),
            pl.BlockSpec((np_, aw), lambda i: (0, 0)),
            pl.BlockSpec((blk, d), lambda i: (i, 0)),
            pl.BlockSpec((d, 1), lambda i: (0, 0)),
            pl.BlockSpec((np_, bw), lambda i: (0, 0)),
        ],
        out_specs=pl.BlockSpec((blk, bw), lambda i: (i, 0)),
        out_shape=jax.ShapeDtypeStruct((np_, bw), jnp.float32),
        interpret=interpret,
    )(idx, a, bm, vt, xt)


_SC_NC = 2
_SC_NS = 16
_SC_L = 16


def _tanh_sc(z):
    ez = jnp.exp(z + z)
    return 1.0 - 2.0 / (ez + 1.0)


def _make_edge_sc(np_, k, d, bw, nb):
    nw = _SC_NC * _SC_NS
    npw = np_ // nw
    groups = npw // nb
    cpr = bw // _SC_L
    dch = d // _SC_L
    mesh = plsc.VectorSubcoreMesh(core_axis_name="c", subcore_axis_name="s")

    @functools.partial(
        pl.kernel, mesh=mesh,
        compiler_params=pltpu.CompilerParams(needs_layout_passes=False),
        out_type=jax.ShapeDtypeStruct((np_, bw), jnp.float32),
        scratch_types=[
            pltpu.VMEM((nb * k,), jnp.int32),
            pltpu.VMEM((nb * k, 2 * d), jnp.float32),
            pltpu.VMEM((nb * k, bw), jnp.float32),
            pltpu.VMEM((nb, d), jnp.float32),
            pltpu.VMEM((nb, bw), jnp.float32),
            pltpu.VMEM((d,), jnp.float32),
            pltpu.SemaphoreType.DMA,
        ],
    )
    def edge_sc(idxf_hbm, a_hbm, bm_hbm, vt_hbm, xt_hbm, out_hbm,
                idx_v, arows_v, xrows_v, bm_v, agg_v, vt_v, sem):
        wid = lax.axis_index("s") * _SC_NC + lax.axis_index("c")
        base = wid * npw
        pltpu.sync_copy(vt_hbm, vt_v)
        lanes = lax.iota(jnp.int32, _SC_L)

        def group_body(g, carry):
            node0 = base + g * nb
            pltpu.sync_copy(idxf_hbm.at[pl.ds(node0 * k, nb * k)], idx_v)
            pltpu.async_copy(a_hbm.at[idx_v], arows_v, sem).wait()
            pltpu.async_copy(xt_hbm.at[idx_v], xrows_v, sem).wait()
            pltpu.sync_copy(bm_hbm.at[pl.ds(node0, nb)], bm_v)

            def node_body(nn, ncarry):
                row0 = nn * k
                bmc = [bm_v[nn, pl.ds(c * _SC_L, _SC_L)] for c in range(dch)]
                vc = [vt_v[pl.ds(c * _SC_L, _SC_L)] for c in range(dch)]
                score = jnp.zeros((_SC_L,), jnp.float32)
                for kk in range(k):
                    part = jnp.zeros((_SC_L,), jnp.float32)
                    for c in range(dch):
                        z = arows_v[row0 + kk,
                                    pl.ds(c * _SC_L, _SC_L)] + bmc[c]
                        part = part + vc[c] * _tanh_sc(z)
                    score = jnp.where(lanes == kk, jnp.sum(part), score)
                m = jnp.max(score)
                e = jnp.exp(score - m)
                attn = e / (jnp.sum(e) + 1e-8)

                accs = [jnp.zeros((_SC_L,), jnp.float32) for _ in range(cpr)]
                for kk in range(k):
                    ak = jnp.sum(jnp.where(lanes == kk, attn, 0.0))
                    for c in range(cpr):
                        accs[c] = accs[c] + ak * xrows_v[
                            row0 + kk, pl.ds(c * _SC_L, _SC_L)]
                for c in range(cpr):
                    agg_v[nn, pl.ds(c * _SC_L, _SC_L)] = accs[c]
                return ncarry

            lax.fori_loop(0, nb, node_body, 0)
            pltpu.sync_copy(agg_v, out_hbm.at[pl.ds(node0, nb)])
            return carry

        lax.fori_loop(0, groups, group_body, 0)

    return edge_sc


def _mlp_body(x_ref, agg_ref, fwx_ref, fwa_ref, fb_ref, w1t_ref, b1_ref,
              w2t_ref, b2_ref, out_ref):
    h = jnp.dot(x_ref[...], fwx_ref[...], preferred_element_type=jnp.float32)
    h = h + jnp.dot(agg_ref[...], fwa_ref[...],
                    preferred_element_type=jnp.float32)
    h = jax.nn.relu(h + fb_ref[...])
    h1 = jax.nn.relu(jnp.dot(h, w1t_ref[...],
                             preferred_element_type=jnp.float32) + b1_ref[...])
    out_ref[...] = jnp.dot(h1, w2t_ref[...],
                           preferred_element_type=jnp.float32) + b2_ref[...]


def _run_mlp(xf, aggf, fwx, fwa, fb, w1t, b1, w2t, b2, blk, interpret=False):
    rows, w = xf.shape
    h = fwx.shape[1]
    h2 = w1t.shape[1]
    grid = (rows // blk,)
    return pl.pallas_call(
        _mlp_body,
        grid=grid,
        in_specs=[
            pl.BlockSpec((blk, w), lambda i: (i, 0)),
            pl.BlockSpec((blk, w), lambda i: (i, 0)),
            pl.BlockSpec((w, h), lambda i: (0, 0)),
            pl.BlockSpec((w, h), lambda i: (0, 0)),
            pl.BlockSpec((1, h), lambda i: (0, 0)),
            pl.BlockSpec((h, h2), lambda i: (0, 0)),
            pl.BlockSpec((1, h2), lambda i: (0, 0)),
            pl.BlockSpec((h2, 1), lambda i: (0, 0)),
            pl.BlockSpec((1, 1), lambda i: (0, 0)),
        ],
        out_specs=pl.BlockSpec((blk, 1), lambda i: (i, 0)),
        out_shape=jax.ShapeDtypeStruct((rows, 1), jnp.float32),
        interpret=interpret,
    )(xf, aggf, fwx, fwa, fb, w1t, b1, w2t, b2)


def _pipeline(x, emb, Wq, Wk, v_w, fc_w, fc_b, w1, b1, w2, b2,
              n, w, d, k, b, blk_topk, blk_edge, chunk, blk_mlp,
              interpret=False):
    np_ = ((n + blk_topk - 1) // blk_topk) * blk_topk
    emb_p = jnp.pad(emb, ((0, np_ - n), (0, 0)))

    idx, a_proj, bm_proj = _run_topk(emb_p, Wq, Wk, n, k, blk_topk,
                                     interpret=interpret)

    xt = jnp.pad(x.transpose(1, 0, 2).reshape(n, b * w), ((0, np_ - n), (0, 0)))
    vt = v_w.reshape(d, 1)

    if interpret:
        aggt = _run_edge_tc(idx, a_proj, bm_proj, vt, xt, k, blk_edge, chunk,
                            interpret=True)
    else:
        edge_sc = _make_edge_sc(np_, k, d, b * w, nb=8)
        aggt = edge_sc(idx.reshape(np_ * k), a_proj, bm_proj,
                       v_w.reshape(d), xt)

    xf = xt.reshape(np_ * b, w)
    aggf = aggt.reshape(np_ * b, w)
    fwx = fc_w[:, :w].T
    fwa = fc_w[:, w:].T
    w1t = w1.T
    w2t = w2.T
    pred = _run_mlp(xf, aggf, fwx, fwa, fc_b[None, :], w1t, b1[None, :], w2t,
                    b2[None, :], blk_mlp, interpret=interpret)
    return pred.reshape(np_, b)[:n, :].T


def kernel(x, emb, Wq, Wk, v_w, fc_w, fc_b, w1, b1, w2, b2):
    return _pipeline(x, emb, Wq, Wk, v_w, fc_w, fc_b, w1, b1, w2, b2,
                     n=_N, w=_W, d=_D, k=_K, b=_B,
                     blk_topk=256, blk_edge=128, chunk=2048, blk_mlp=512)


# speedup vs baseline: 12.5752x; 1.1725x over previous
import functools

import jax
import jax.numpy as jnp
from jax import lax
from jax.experimental import pallas as pl
from jax.experimental.pallas import tpu as pltpu
from jax.experimental.pallas import tpu_sc as plsc

_N = 10000
_W = 128
_D = 64
_H = 64
_K = 16
_B = 4

_NEG = -3.4e38


def _topk_body(n_real, k, blk, row_base, emb_ref, idx_ref, e_scr):
    pid = pl.program_id(0)

    @pl.when(pid == 0)
    def _():
        e = emb_ref[...]
        nrm = jnp.sqrt(jnp.sum(e * e, axis=1, keepdims=True)) + 1e-8
        e_scr[...] = e / nrm

    e_all = e_scr[...]
    rows = e_scr[pl.ds(row_base + pid * blk, blk), :]
    sim = jax.lax.dot_general(
        rows, e_all, (((1,), (1,)), ((), ())),
        preferred_element_type=jnp.float32)

    np_cols = e_all.shape[0]
    col = jax.lax.broadcasted_iota(jnp.int32, (blk, np_cols), 1)
    rowid = (row_base + pid * blk
             + jax.lax.broadcasted_iota(jnp.int32, (blk, np_cols), 0))
    sim = jnp.where(col == rowid, sim - 1e9, sim)
    sim = jnp.where(col >= n_real, _NEG, sim)

    outs = []
    for _ in range(k):
        idx_t = jnp.argmax(sim, axis=1).astype(jnp.int32)
        outs.append(idx_t)
        sim = jnp.where(col == idx_t[:, None], _NEG, sim)
    idx_ref[...] = jnp.stack(outs, axis=1)


def _run_topk(emb_p, n_real, k, blk, row_base, n_rows, interpret=False):
    np_, d = emb_p.shape
    grid = (n_rows // blk,)
    return pl.pallas_call(
        functools.partial(_topk_body, n_real, k, blk, row_base),
        grid=grid,
        in_specs=[
            pl.BlockSpec((np_, d), lambda i: (0, 0)),
        ],
        out_specs=pl.BlockSpec((blk, k), lambda i: (i, 0)),
        out_shape=jax.ShapeDtypeStruct((n_rows, k), jnp.int32),
        scratch_shapes=[pltpu.VMEM((np_, d), jnp.float32)],
        interpret=interpret,
    )(emb_p)


def _proj_body(emb_ref, wq_ref, wk_ref, a_ref, bm_ref):
    e = emb_ref[...]
    aq = jax.lax.dot_general(
        e, wq_ref[...], (((1,), (1,)), ((), ())),
        preferred_element_type=jnp.float32)
    a_ref[...] = jnp.concatenate([aq, jnp.zeros_like(aq)], axis=1)
    bm_ref[...] = jax.lax.dot_general(
        e, wk_ref[...], (((1,), (1,)), ((), ())),
        preferred_element_type=jnp.float32)


def _run_proj(emb_p, wq, wk, interpret=False):
    np_, d = emb_p.shape
    return pl.pallas_call(
        _proj_body,
        out_shape=[
            jax.ShapeDtypeStruct((np_, 2 * d), jnp.float32),
            jax.ShapeDtypeStruct((np_, d), jnp.float32),
        ],
        interpret=interpret,
    )(emb_p, wq, wk)


def _edge_body(k, blk, chunk, idx_ref, a_ref, bm_ref, vt_ref, xt_ref, agg_ref):
    np_cols = a_ref.shape[0]
    aw = a_ref.shape[1]
    d = bm_ref.shape[1]
    bw = xt_ref.shape[1]
    idxb = idx_ref[...]
    bmb = bm_ref[...]
    vt = vt_ref[...]

    scores = []
    for t in range(k):
        tgt = idxb[:, t][:, None]
        ak = jnp.zeros((blk, aw), dtype=jnp.float32)
        for c in range(0, np_cols, chunk):
            cio = c + jax.lax.broadcasted_iota(jnp.int32, (blk, chunk), 1)
            oh = (cio == tgt).astype(jnp.float32)
            ak = ak + jnp.dot(oh, a_ref[pl.ds(c, chunk), :],
                              preferred_element_type=jnp.float32)
        z = jnp.tanh(ak[:, :d] + bmb)
        scores.append(jnp.dot(z, vt, preferred_element_type=jnp.float32)[:, 0])
    s = jnp.stack(scores, axis=1)

    mx = jnp.max(s, axis=1, keepdims=True)
    e = jnp.exp(s - mx)
    attn = e / (jnp.sum(e, axis=1, keepdims=True) + 1e-8)

    acc = jnp.zeros((blk, bw), dtype=jnp.float32)
    for c in range(0, np_cols, chunk):
        cio = c + jax.lax.broadcasted_iota(jnp.int32, (blk, chunk), 1)
        p = jnp.zeros((blk, chunk), dtype=jnp.float32)
        for t in range(k):
            p = p + jnp.where(cio == idxb[:, t][:, None],
                              attn[:, t][:, None], 0.0)
        acc = acc + jnp.dot(p, xt_ref[pl.ds(c, chunk), :],
                            preferred_element_type=jnp.float32)
    agg_ref[...] = acc


def _run_edge_tc(idx, a, bm, vt, xt, k, blk, chunk, interpret=False):
    np_, aw = a.shape
    d = bm.shape[1]
    bw = xt.shape[1]
    grid = (np_ // blk,)
    return pl.pallas_call(
        functools.partial(_edge_body, k, blk, chunk),
        grid=grid,
        in_specs=[
            pl.BlockSpec((blk, k), lambda i: (i, 0)),
            pl.BlockSpec((np_, aw), lambda i: (0, 0)),
            pl.BlockSpec((blk, d), lambda i: (i, 0)),
            pl.BlockSpec((d, 1), lambda i: (0, 0)),
            pl.BlockSpec((np_, bw), lambda i: (0, 0)),
        ],
        out_specs=pl.BlockSpec((blk, bw), lambda i: (i, 0)),
        out_shape=jax.ShapeDtypeStruct((np_, bw), jnp.float32),
        interpret=interpret,
    )(idx, a, bm, vt, xt)


_SC_NC = 2
_SC_NS = 16
_SC_L = 16


def _tanh_sc(z):
    ez = jnp.exp(z + z)
    return 1.0 - 2.0 / (ez + 1.0)


def _make_edge_sc(np_tab, nrows, k, d, bw, nb):
    nw = _SC_NC * _SC_NS
    npw = nrows // nw
    groups = npw // nb
    cpr = bw // _SC_L
    dch = d // _SC_L
    mesh = plsc.VectorSubcoreMesh(core_axis_name="c", subcore_axis_name="s")

    @functools.partial(
        pl.kernel, mesh=mesh,
        compiler_params=pltpu.CompilerParams(needs_layout_passes=False),
        out_type=jax.ShapeDtypeStruct((nrows, bw), jnp.float32),
        scratch_types=[
            pltpu.VMEM((nb * k,), jnp.int32),
            pltpu.VMEM((nb * k, 2 * d), jnp.float32),
            pltpu.VMEM((nb * k, bw), jnp.float32),
            pltpu.VMEM((nb, d), jnp.float32),
            pltpu.VMEM((nb, bw), jnp.float32),
            pltpu.VMEM((d,), jnp.float32),
            pltpu.SemaphoreType.DMA,
        ],
    )
    def edge_sc(idxf_hbm, a_hbm, bm_hbm, vt_hbm, xt_hbm, out_hbm,
                idx_v, arows_v, xrows_v, bm_v, agg_v, vt_v, sem):
        wid = lax.axis_index("s") * _SC_NC + lax.axis_index("c")
        base = wid * npw
        pltpu.sync_copy(vt_hbm, vt_v)
        lanes = lax.iota(jnp.int32, _SC_L)

        def group_body(g, carry):
            node0 = base + g * nb
            pltpu.sync_copy(idxf_hbm.at[pl.ds(node0 * k, nb * k)], idx_v)
            pltpu.async_copy(a_hbm.at[idx_v], arows_v, sem).wait()
            pltpu.async_copy(xt_hbm.at[idx_v], xrows_v, sem).wait()
            pltpu.sync_copy(bm_hbm.at[pl.ds(node0, nb)], bm_v)

            def node_body(nn, ncarry):
                row0 = nn * k
                bmc = [bm_v[nn, pl.ds(c * _SC_L, _SC_L)] for c in range(dch)]
                vc = [vt_v[pl.ds(c * _SC_L, _SC_L)] for c in range(dch)]
                score = jnp.zeros((_SC_L,), jnp.float32)
                for kk in range(k):
                    part = jnp.zeros((_SC_L,), jnp.float32)
                    for c in range(dch):
                        z = arows_v[row0 + kk,
                                    pl.ds(c * _SC_L, _SC_L)] + bmc[c]
                        part = part + vc[c] * _tanh_sc(z)
                    score = jnp.where(lanes == kk, jnp.sum(part), score)
                m = jnp.max(score)
                e = jnp.exp(score - m)
                attn = e / (jnp.sum(e) + 1e-8)

                accs = [jnp.zeros((_SC_L,), jnp.float32) for _ in range(cpr)]
                for kk in range(k):
                    ak = jnp.sum(jnp.where(lanes == kk, attn, 0.0))
                    for c in range(cpr):
                        accs[c] = accs[c] + ak * xrows_v[
                            row0 + kk, pl.ds(c * _SC_L, _SC_L)]
                for c in range(cpr):
                    agg_v[nn, pl.ds(c * _SC_L, _SC_L)] = accs[c]
                return ncarry

            lax.fori_loop(0, nb, node_body, 0)
            pltpu.sync_copy(agg_v, out_hbm.at[pl.ds(node0, nb)])
            return carry

        lax.fori_loop(0, groups, group_body, 0)

    return edge_sc


def _mlp_body(x_ref, agg_ref, fwx_ref, fwa_ref, fb_ref, w1t_ref, b1_ref,
              w2t_ref, b2_ref, out_ref):
    h = jnp.dot(x_ref[...], fwx_ref[...], preferred_element_type=jnp.float32)
    h = h + jnp.dot(agg_ref[...], fwa_ref[...],
                    preferred_element_type=jnp.float32)
    h = jax.nn.relu(h + fb_ref[...])
    h1 = jax.nn.relu(jnp.dot(h, w1t_ref[...],
                             preferred_element_type=jnp.float32) + b1_ref[...])
    out_ref[...] = jnp.dot(h1, w2t_ref[...],
                           preferred_element_type=jnp.float32) + b2_ref[...]


def _run_mlp(xf, aggf, fwx, fwa, fb, w1t, b1, w2t, b2, blk, interpret=False):
    rows, w = xf.shape
    h = fwx.shape[1]
    h2 = w1t.shape[1]
    grid = (rows // blk,)
    return pl.pallas_call(
        _mlp_body,
        grid=grid,
        in_specs=[
            pl.BlockSpec((blk, w), lambda i: (i, 0)),
            pl.BlockSpec((blk, w), lambda i: (i, 0)),
            pl.BlockSpec((w, h), lambda i: (0, 0)),
            pl.BlockSpec((w, h), lambda i: (0, 0)),
            pl.BlockSpec((1, h), lambda i: (0, 0)),
            pl.BlockSpec((h, h2), lambda i: (0, 0)),
            pl.BlockSpec((1, h2), lambda i: (0, 0)),
            pl.BlockSpec((h2, 1), lambda i: (0, 0)),
            pl.BlockSpec((1, 1), lambda i: (0, 0)),
        ],
        out_specs=pl.BlockSpec((blk, 1), lambda i: (i, 0)),
        out_shape=jax.ShapeDtypeStruct((rows, 1), jnp.float32),
        interpret=interpret,
    )(xf, aggf, fwx, fwa, fb, w1t, b1, w2t, b2)


def _pipeline(x, emb, Wq, Wk, v_w, fc_w, fc_b, w1, b1, w2, b2,
              n, w, d, k, b, blk_topk, blk_edge, chunk, blk_mlp,
              interpret=False):
    np_ = ((n + 2 * blk_topk - 1) // (2 * blk_topk)) * (2 * blk_topk)
    half = np_ // 2
    emb_p = jnp.pad(emb, ((0, np_ - n), (0, 0)))

    a_proj, bm_proj = _run_proj(emb_p, Wq, Wk, interpret=interpret)
    idx1 = _run_topk(emb_p, n, k, blk_topk, 0, half, interpret=interpret)
    idx2 = _run_topk(emb_p, n, k, blk_topk, half, half, interpret=interpret)

    xt = jnp.pad(x.transpose(1, 0, 2).reshape(n, b * w), ((0, np_ - n), (0, 0)))
    vt = v_w.reshape(d, 1)

    fwx = fc_w[:, :w].T
    fwa = fc_w[:, w:].T
    w1t = w1.T
    w2t = w2.T

    if interpret:
        idx = jnp.concatenate([idx1, idx2], axis=0)
        aggt = _run_edge_tc(idx, a_proj, bm_proj, vt, xt, k, blk_edge, chunk,
                            interpret=True)
        aggs = [aggt[:half], aggt[half:]]
    else:
        edge_sc = _make_edge_sc(np_, half, k, d, b * w, nb=8)
        aggs = [
            edge_sc(idx1.reshape(half * k), a_proj, bm_proj[:half],
                    v_w.reshape(d), xt),
            edge_sc(idx2.reshape(half * k), a_proj, bm_proj[half:],
                    v_w.reshape(d), xt),
        ]

    preds = []
    for hh in range(2):
        xf = xt[hh * half:(hh + 1) * half].reshape(half * b, w)
        aggf = aggs[hh].reshape(half * b, w)
        preds.append(_run_mlp(xf, aggf, fwx, fwa, fc_b[None, :], w1t,
                              b1[None, :], w2t, b2[None, :], blk_mlp,
                              interpret=interpret).reshape(half, b))
    return jnp.concatenate(preds, axis=0)[:n, :].T


def kernel(x, emb, Wq, Wk, v_w, fc_w, fc_b, w1, b1, w2, b2):
    return _pipeline(x, emb, Wq, Wk, v_w, fc_w, fc_b, w1, b1, w2, b2,
                     n=_N, w=_W, d=_D, k=_K, b=_B,
                     blk_topk=256, blk_edge=128, chunk=2048, blk_mlp=512)


# speedup vs baseline: 13.4894x; 1.0727x over previous
import functools

import jax
import jax.numpy as jnp
from jax import lax
from jax.experimental import pallas as pl
from jax.experimental.pallas import tpu as pltpu
from jax.experimental.pallas import tpu_sc as plsc

_N = 10000
_W = 128
_D = 64
_H = 64
_K = 16
_B = 4

_NEG = -3.4e38


def _topk_body(n_real, k, blk, row_base, emb_ref, idx_ref, e_scr):
    pid = pl.program_id(0)

    @pl.when(pid == 0)
    def _():
        e = emb_ref[...]
        nrm = jnp.sqrt(jnp.sum(e * e, axis=1, keepdims=True)) + 1e-8
        e_scr[...] = e / nrm

    e_all = e_scr[...]
    rows = e_scr[pl.ds(row_base + pid * blk, blk), :]
    sim = jax.lax.dot_general(
        rows, e_all, (((1,), (1,)), ((), ())),
        preferred_element_type=jnp.float32)

    np_cols = e_all.shape[0]
    col = jax.lax.broadcasted_iota(jnp.int32, (blk, np_cols), 1)
    rowid = (row_base + pid * blk
             + jax.lax.broadcasted_iota(jnp.int32, (blk, np_cols), 0))
    sim = jnp.where(col == rowid, sim - 1e9, sim)
    sim = jnp.where(col >= n_real, _NEG, sim)

    outs = []
    for _ in range(k):
        idx_t = jnp.argmax(sim, axis=1).astype(jnp.int32)
        outs.append(idx_t)
        sim = jnp.where(col == idx_t[:, None], _NEG, sim)
    idx_ref[...] = jnp.stack(outs, axis=1)


def _run_topk(emb_p, n_real, k, blk, row_base, n_rows, interpret=False):
    np_, d = emb_p.shape
    grid = (n_rows // blk,)
    return pl.pallas_call(
        functools.partial(_topk_body, n_real, k, blk, row_base),
        grid=grid,
        in_specs=[
            pl.BlockSpec((np_, d), lambda i: (0, 0)),
        ],
        out_specs=pl.BlockSpec((blk, k), lambda i: (i, 0)),
        out_shape=jax.ShapeDtypeStruct((n_rows, k), jnp.int32),
        scratch_shapes=[pltpu.VMEM((np_, d), jnp.float32)],
        interpret=interpret,
    )(emb_p)


def _proj_body(emb_ref, wq_ref, wk_ref, a_ref, bm_ref):
    e = emb_ref[...]
    aq = jax.lax.dot_general(
        e, wq_ref[...], (((1,), (1,)), ((), ())),
        preferred_element_type=jnp.float32)
    a_ref[...] = jnp.concatenate([aq, jnp.zeros_like(aq)], axis=1)
    bm_ref[...] = jax.lax.dot_general(
        e, wk_ref[...], (((1,), (1,)), ((), ())),
        preferred_element_type=jnp.float32)


def _run_proj(emb_p, wq, wk, interpret=False):
    np_, d = emb_p.shape
    return pl.pallas_call(
        _proj_body,
        out_shape=[
            jax.ShapeDtypeStruct((np_, 2 * d), jnp.float32),
            jax.ShapeDtypeStruct((np_, d), jnp.float32),
        ],
        interpret=interpret,
    )(emb_p, wq, wk)


def _edge_body(k, blk, chunk, idx_ref, a_ref, bm_ref, vt_ref, xt_ref, agg_ref):
    np_cols = a_ref.shape[0]
    aw = a_ref.shape[1]
    d = bm_ref.shape[1]
    bw = xt_ref.shape[1]
    idxb = idx_ref[...]
    bmb = bm_ref[...]
    vt = vt_ref[...]

    scores = []
    for t in range(k):
        tgt = idxb[:, t][:, None]
        ak = jnp.zeros((blk, aw), dtype=jnp.float32)
        for c in range(0, np_cols, chunk):
            cio = c + jax.lax.broadcasted_iota(jnp.int32, (blk, chunk), 1)
            oh = (cio == tgt).astype(jnp.float32)
            ak = ak + jnp.dot(oh, a_ref[pl.ds(c, chunk), :],
                              preferred_element_type=jnp.float32)
        z = jnp.tanh(ak[:, :d] + bmb)
        scores.append(jnp.dot(z, vt, preferred_element_type=jnp.float32)[:, 0])
    s = jnp.stack(scores, axis=1)

    mx = jnp.max(s, axis=1, keepdims=True)
    e = jnp.exp(s - mx)
    attn = e / (jnp.sum(e, axis=1, keepdims=True) + 1e-8)

    acc = jnp.zeros((blk, bw), dtype=jnp.float32)
    for c in range(0, np_cols, chunk):
        cio = c + jax.lax.broadcasted_iota(jnp.int32, (blk, chunk), 1)
        p = jnp.zeros((blk, chunk), dtype=jnp.float32)
        for t in range(k):
            p = p + jnp.where(cio == idxb[:, t][:, None],
                              attn[:, t][:, None], 0.0)
        acc = acc + jnp.dot(p, xt_ref[pl.ds(c, chunk), :],
                            preferred_element_type=jnp.float32)
    agg_ref[...] = acc


def _run_edge_tc(idx, a, bm, vt, xt, k, blk, chunk, interpret=False):
    np_, aw = a.shape
    d = bm.shape[1]
    bw = xt.shape[1]
    grid = (np_ // blk,)
    return pl.pallas_call(
        functools.partial(_edge_body, k, blk, chunk),
        grid=grid,
        in_specs=[
            pl.BlockSpec((blk, k), lambda i: (i, 0)),
            pl.BlockSpec((np_, aw), lambda i: (0, 0)),
            pl.BlockSpec((blk, d), lambda i: (i, 0)),
            pl.BlockSpec((d, 1), lambda i: (0, 0)),
            pl.BlockSpec((np_, bw), lambda i: (0, 0)),
        ],
        out_specs=pl.BlockSpec((blk, bw), lambda i: (i, 0)),
        out_shape=jax.ShapeDtypeStruct((np_, bw), jnp.float32),
        interpret=interpret,
    )(idx, a, bm, vt, xt)


_SC_NC = 2
_SC_NS = 16
_SC_L = 16


def _tanh_sc(z):
    ez = jnp.exp(z + z)
    return 1.0 - 2.0 / (ez + 1.0)


def _make_edge_sc(np_tab, nrows, k, d, bw, nb):
    nw = _SC_NC * _SC_NS
    npw = nrows // nw
    groups = npw // nb
    cpr = bw // _SC_L
    dch = d // _SC_L
    mesh = plsc.VectorSubcoreMesh(core_axis_name="c", subcore_axis_name="s")

    @functools.partial(
        pl.kernel, mesh=mesh,
        compiler_params=pltpu.CompilerParams(needs_layout_passes=False),
        out_type=jax.ShapeDtypeStruct((nrows, bw), jnp.float32),
        scratch_types=[
            pltpu.VMEM((nb * k,), jnp.int32),
            pltpu.VMEM((nb * k, 2 * d), jnp.float32),
            pltpu.VMEM((nb * k, bw), jnp.float32),
            pltpu.VMEM((nb, d), jnp.float32),
            pltpu.VMEM((nb, bw), jnp.float32),
            pltpu.VMEM((d,), jnp.float32),
            pltpu.SemaphoreType.DMA,
        ],
    )
    def edge_sc(idxf_hbm, a_hbm, bm_hbm, vt_hbm, xt_hbm, out_hbm,
                idx_v, arows_v, xrows_v, bm_v, agg_v, vt_v, sem):
        wid = lax.axis_index("s") * _SC_NC + lax.axis_index("c")
        base = wid * npw
        pltpu.sync_copy(vt_hbm, vt_v)
        lanes = lax.iota(jnp.int32, _SC_L)

        def group_body(g, carry):
            node0 = base + g * nb
            pltpu.sync_copy(idxf_hbm.at[pl.ds(node0 * k, nb * k)], idx_v)
            pltpu.async_copy(a_hbm.at[idx_v], arows_v, sem).wait()
            pltpu.async_copy(xt_hbm.at[idx_v], xrows_v, sem).wait()
            pltpu.sync_copy(bm_hbm.at[pl.ds(node0, nb)], bm_v)

            def node_body(nn, ncarry):
                row0 = nn * k
                bmc = [bm_v[nn, pl.ds(c * _SC_L, _SC_L)] for c in range(dch)]
                vc = [vt_v[pl.ds(c * _SC_L, _SC_L)] for c in range(dch)]
                score = jnp.zeros((_SC_L,), jnp.float32)
                for kk in range(k):
                    part = jnp.zeros((_SC_L,), jnp.float32)
                    for c in range(dch):
                        z = arows_v[row0 + kk,
                                    pl.ds(c * _SC_L, _SC_L)] + bmc[c]
                        part = part + vc[c] * _tanh_sc(z)
                    score = jnp.where(lanes == kk, jnp.sum(part), score)
                m = jnp.max(score)
                e = jnp.exp(score - m)
                attn = e / (jnp.sum(e) + 1e-8)

                accs = [jnp.zeros((_SC_L,), jnp.float32) for _ in range(cpr)]
                for kk in range(k):
                    ak = jnp.sum(jnp.where(lanes == kk, attn, 0.0))
                    for c in range(cpr):
                        accs[c] = accs[c] + ak * xrows_v[
                            row0 + kk, pl.ds(c * _SC_L, _SC_L)]
                for c in range(cpr):
                    agg_v[nn, pl.ds(c * _SC_L, _SC_L)] = accs[c]
                return ncarry

            lax.fori_loop(0, nb, node_body, 0)
            pltpu.sync_copy(agg_v, out_hbm.at[pl.ds(node0, nb)])
            return carry

        lax.fori_loop(0, groups, group_body, 0)

    return edge_sc


def _mlp_body(x_ref, agg_ref, fwx_ref, fwa_ref, fb_ref, w1t_ref, b1_ref,
              w2t_ref, b2_ref, out_ref):
    h = jnp.dot(x_ref[...], fwx_ref[...], preferred_element_type=jnp.float32)
    h = h + jnp.dot(agg_ref[...], fwa_ref[...],
                    preferred_element_type=jnp.float32)
    h = jax.nn.relu(h + fb_ref[...])
    h1 = jax.nn.relu(jnp.dot(h, w1t_ref[...],
                             preferred_element_type=jnp.float32) + b1_ref[...])
    out_ref[...] = jnp.dot(h1, w2t_ref[...],
                           preferred_element_type=jnp.float32) + b2_ref[...]


def _run_mlp(xf, aggf, fwx, fwa, fb, w1t, b1, w2t, b2, blk, interpret=False):
    rows, w = xf.shape
    h = fwx.shape[1]
    h2 = w1t.shape[1]
    grid = (rows // blk,)
    return pl.pallas_call(
        _mlp_body,
        grid=grid,
        in_specs=[
            pl.BlockSpec((blk, w), lambda i: (i, 0)),
            pl.BlockSpec((blk, w), lambda i: (i, 0)),
            pl.BlockSpec((w, h), lambda i: (0, 0)),
            pl.BlockSpec((w, h), lambda i: (0, 0)),
            pl.BlockSpec((1, h), lambda i: (0, 0)),
            pl.BlockSpec((h, h2), lambda i: (0, 0)),
            pl.BlockSpec((1, h2), lambda i: (0, 0)),
            pl.BlockSpec((h2, 1), lambda i: (0, 0)),
            pl.BlockSpec((1, 1), lambda i: (0, 0)),
        ],
        out_specs=pl.BlockSpec((blk, 1), lambda i: (i, 0)),
        out_shape=jax.ShapeDtypeStruct((rows, 1), jnp.float32),
        interpret=interpret,
    )(xf, aggf, fwx, fwa, fb, w1t, b1, w2t, b2)


def _pipeline(x, emb, Wq, Wk, v_w, fc_w, fc_b, w1, b1, w2, b2,
              n, w, d, k, b, blk_topk, blk_edge, chunk, blk_mlp,
              interpret=False):
    ns = 4
    np_ = ((n + ns * blk_topk - 1) // (ns * blk_topk)) * (ns * blk_topk)
    half = np_ // ns
    emb_p = jnp.pad(emb, ((0, np_ - n), (0, 0)))

    a_proj, bm_proj = _run_proj(emb_p, Wq, Wk, interpret=interpret)
    idxs = [_run_topk(emb_p, n, k, blk_topk, hh * half, half,
                      interpret=interpret) for hh in range(ns)]

    xt = jnp.pad(x.transpose(1, 0, 2).reshape(n, b * w), ((0, np_ - n), (0, 0)))
    vt = v_w.reshape(d, 1)

    fwx = fc_w[:, :w].T
    fwa = fc_w[:, w:].T
    w1t = w1.T
    w2t = w2.T

    if interpret:
        idx = jnp.concatenate(idxs, axis=0)
        aggt = _run_edge_tc(idx, a_proj, bm_proj, vt, xt, k, blk_edge, chunk,
                            interpret=True)
        aggs = [aggt[hh * half:(hh + 1) * half] for hh in range(ns)]
    else:
        edge_sc = _make_edge_sc(np_, half, k, d, b * w, nb=8)
        aggs = [
            edge_sc(idxs[hh].reshape(half * k), a_proj,
                    bm_proj[hh * half:(hh + 1) * half], v_w.reshape(d), xt)
            for hh in range(ns)
        ]

    preds = []
    for hh in range(ns):
        xf = xt[hh * half:(hh + 1) * half].reshape(half * b, w)
        aggf = aggs[hh].reshape(half * b, w)
        preds.append(_run_mlp(xf, aggf, fwx, fwa, fc_b[None, :], w1t,
                              b1[None, :], w2t, b2[None, :], blk_mlp,
                              interpret=interpret).reshape(half, b))
    return jnp.concatenate(preds, axis=0)[:n, :].T


def kernel(x, emb, Wq, Wk, v_w, fc_w, fc_b, w1, b1, w2, b2):
    return _pipeline(x, emb, Wq, Wk, v_w, fc_w, fc_b, w1, b1, w2, b2,
                     n=_N, w=_W, d=_D, k=_K, b=_B,
                     blk_topk=256, blk_edge=128, chunk=2048, blk_mlp=512)


# speedup vs baseline: 14.5008x; 1.0750x over previous
import functools

import jax
import jax.numpy as jnp
from jax import lax
from jax.experimental import pallas as pl
from jax.experimental.pallas import tpu as pltpu
from jax.experimental.pallas import tpu_sc as plsc

_N = 10000
_W = 128
_D = 64
_H = 64
_K = 16
_B = 4

_NEG = -3.4e38


def _topk_body(n_real, k, blk, row_base, emb_ref, idx_ref, e_scr):
    pid = pl.program_id(0)

    @pl.when(pid == 0)
    def _():
        e = emb_ref[...]
        nrm = jnp.sqrt(jnp.sum(e * e, axis=1, keepdims=True)) + 1e-8
        e_scr[...] = e / nrm

    e_all = e_scr[...]
    rows = e_scr[pl.ds(row_base + pid * blk, blk), :]
    sim = jax.lax.dot_general(
        rows, e_all, (((1,), (1,)), ((), ())),
        preferred_element_type=jnp.float32)

    np_cols = e_all.shape[0]
    col = jax.lax.broadcasted_iota(jnp.int32, (blk, np_cols), 1)
    rowid = (row_base + pid * blk
             + jax.lax.broadcasted_iota(jnp.int32, (blk, np_cols), 0))
    sim = jnp.where(col == rowid, sim - 1e9, sim)
    sim = jnp.where(col >= n_real, _NEG, sim)

    outs = []
    for _ in range(k):
        idx_t = jnp.argmax(sim, axis=1).astype(jnp.int32)
        outs.append(idx_t)
        sim = jnp.where(col == idx_t[:, None], _NEG, sim)
    idx_ref[...] = jnp.stack(outs, axis=1)


def _run_topk(emb_p, n_real, k, blk, row_base, n_rows, interpret=False):
    np_, d = emb_p.shape
    grid = (n_rows // blk,)
    return pl.pallas_call(
        functools.partial(_topk_body, n_real, k, blk, row_base),
        grid=grid,
        in_specs=[
            pl.BlockSpec((np_, d), lambda i: (0, 0)),
        ],
        out_specs=pl.BlockSpec((blk, k), lambda i: (i, 0)),
        out_shape=jax.ShapeDtypeStruct((n_rows, k), jnp.int32),
        scratch_shapes=[pltpu.VMEM((np_, d), jnp.float32)],
        interpret=interpret,
    )(emb_p)


def _proj_body(emb_ref, wq_ref, wk_ref, a_ref, bm_ref):
    e = emb_ref[...]
    aq = jax.lax.dot_general(
        e, wq_ref[...], (((1,), (1,)), ((), ())),
        preferred_element_type=jnp.float32)
    a_ref[...] = jnp.concatenate([aq, jnp.zeros_like(aq)], axis=1)
    bm_ref[...] = jax.lax.dot_general(
        e, wk_ref[...], (((1,), (1,)), ((), ())),
        preferred_element_type=jnp.float32)


def _run_proj(emb_p, wq, wk, interpret=False):
    np_, d = emb_p.shape
    return pl.pallas_call(
        _proj_body,
        out_shape=[
            jax.ShapeDtypeStruct((np_, 2 * d), jnp.float32),
            jax.ShapeDtypeStruct((np_, d), jnp.float32),
        ],
        interpret=interpret,
    )(emb_p, wq, wk)


def _edge_body(k, blk, chunk, idx_ref, a_ref, bm_ref, vt_ref, xt_ref, agg_ref):
    np_cols = a_ref.shape[0]
    aw = a_ref.shape[1]
    d = bm_ref.shape[1]
    bw = xt_ref.shape[1]
    idxb = idx_ref[...]
    bmb = bm_ref[...]
    vt = vt_ref[...]

    scores = []
    for t in range(k):
        tgt = idxb[:, t][:, None]
        ak = jnp.zeros((blk, aw), dtype=jnp.float32)
        for c in range(0, np_cols, chunk):
            cio = c + jax.lax.broadcasted_iota(jnp.int32, (blk, chunk), 1)
            oh = (cio == tgt).astype(jnp.float32)
            ak = ak + jnp.dot(oh, a_ref[pl.ds(c, chunk), :],
                              preferred_element_type=jnp.float32)
        z = jnp.tanh(ak[:, :d] + bmb)
        scores.append(jnp.dot(z, vt, preferred_element_type=jnp.float32)[:, 0])
    s = jnp.stack(scores, axis=1)

    mx = jnp.max(s, axis=1, keepdims=True)
    e = jnp.exp(s - mx)
    attn = e / (jnp.sum(e, axis=1, keepdims=True) + 1e-8)

    acc = jnp.zeros((blk, bw), dtype=jnp.float32)
    for c in range(0, np_cols, chunk):
        cio = c + jax.lax.broadcasted_iota(jnp.int32, (blk, chunk), 1)
        p = jnp.zeros((blk, chunk), dtype=jnp.float32)
        for t in range(k):
            p = p + jnp.where(cio == idxb[:, t][:, None],
                              attn[:, t][:, None], 0.0)
        acc = acc + jnp.dot(p, xt_ref[pl.ds(c, chunk), :],
                            preferred_element_type=jnp.float32)
    agg_ref[...] = acc


def _run_edge_tc(idx, a, bm, vt, xt, k, blk, chunk, interpret=False):
    np_, aw = a.shape
    d = bm.shape[1]
    bw = xt.shape[1]
    grid = (np_ // blk,)
    return pl.pallas_call(
        functools.partial(_edge_body, k, blk, chunk),
        grid=grid,
        in_specs=[
            pl.BlockSpec((blk, k), lambda i: (i, 0)),
            pl.BlockSpec((np_, aw), lambda i: (0, 0)),
            pl.BlockSpec((blk, d), lambda i: (i, 0)),
            pl.BlockSpec((d, 1), lambda i: (0, 0)),
            pl.BlockSpec((np_, bw), lambda i: (0, 0)),
        ],
        out_specs=pl.BlockSpec((blk, bw), lambda i: (i, 0)),
        out_shape=jax.ShapeDtypeStruct((np_, bw), jnp.float32),
        interpret=interpret,
    )(idx, a, bm, vt, xt)


_SC_NC = 2
_SC_NS = 16
_SC_L = 16


def _tanh_sc(z):
    ez = jnp.exp(z + z)
    return 1.0 - 2.0 / (ez + 1.0)


def _make_edge_sc(np_tab, nrows, k, d, bw, nb):
    nw = _SC_NC * _SC_NS
    npw = nrows // nw
    groups = npw // nb
    cpr = bw // _SC_L
    dch = d // _SC_L
    mesh = plsc.VectorSubcoreMesh(core_axis_name="c", subcore_axis_name="s")

    @functools.partial(
        pl.kernel, mesh=mesh,
        compiler_params=pltpu.CompilerParams(needs_layout_passes=False),
        out_type=jax.ShapeDtypeStruct((nrows, bw), jnp.float32),
        scratch_types=[
            pltpu.VMEM((nb * k,), jnp.int32),
            pltpu.VMEM((nb * k, 2 * d), jnp.float32),
            pltpu.VMEM((nb * k, bw), jnp.float32),
            pltpu.VMEM((nb, d), jnp.float32),
            pltpu.VMEM((nb, bw), jnp.float32),
            pltpu.VMEM((d,), jnp.float32),
            pltpu.SemaphoreType.DMA,
        ],
    )
    def edge_sc(idxf_hbm, a_hbm, bm_hbm, vt_hbm, xt_hbm, out_hbm,
                idx_v, arows_v, xrows_v, bm_v, agg_v, vt_v, sem):
        wid = lax.axis_index("s") * _SC_NC + lax.axis_index("c")
        base = wid * npw
        pltpu.sync_copy(vt_hbm, vt_v)
        lanes = lax.iota(jnp.int32, _SC_L)

        def group_body(g, carry):
            node0 = base + g * nb
            pltpu.sync_copy(idxf_hbm.at[pl.ds(node0 * k, nb * k)], idx_v)
            pltpu.async_copy(a_hbm.at[idx_v], arows_v, sem).wait()
            pltpu.async_copy(xt_hbm.at[idx_v], xrows_v, sem).wait()
            pltpu.sync_copy(bm_hbm.at[pl.ds(node0, nb)], bm_v)

            def node_body(nn, ncarry):
                row0 = nn * k
                bmc = [bm_v[nn, pl.ds(c * _SC_L, _SC_L)] for c in range(dch)]
                vc = [vt_v[pl.ds(c * _SC_L, _SC_L)] for c in range(dch)]
                score = jnp.zeros((_SC_L,), jnp.float32)
                for kk in range(k):
                    part = jnp.zeros((_SC_L,), jnp.float32)
                    for c in range(dch):
                        z = arows_v[row0 + kk,
                                    pl.ds(c * _SC_L, _SC_L)] + bmc[c]
                        part = part + vc[c] * _tanh_sc(z)
                    score = jnp.where(lanes == kk, jnp.sum(part), score)
                m = jnp.max(score)
                e = jnp.exp(score - m)
                attn = e / (jnp.sum(e) + 1e-8)

                accs = [jnp.zeros((_SC_L,), jnp.float32) for _ in range(cpr)]
                for kk in range(k):
                    ak = jnp.sum(jnp.where(lanes == kk, attn, 0.0))
                    for c in range(cpr):
                        accs[c] = accs[c] + ak * xrows_v[
                            row0 + kk, pl.ds(c * _SC_L, _SC_L)]
                for c in range(cpr):
                    agg_v[nn, pl.ds(c * _SC_L, _SC_L)] = accs[c]
                return ncarry

            lax.fori_loop(0, nb, node_body, 0)
            pltpu.sync_copy(agg_v, out_hbm.at[pl.ds(node0, nb)])
            return carry

        lax.fori_loop(0, groups, group_body, 0)

    return edge_sc


def _mlp_body(x_ref, agg_ref, fwx_ref, fwa_ref, fb_ref, w1t_ref, b1_ref,
              w2t_ref, b2_ref, out_ref):
    h = jnp.dot(x_ref[...], fwx_ref[...], preferred_element_type=jnp.float32)
    h = h + jnp.dot(agg_ref[...], fwa_ref[...],
                    preferred_element_type=jnp.float32)
    h = jax.nn.relu(h + fb_ref[...])
    h1 = jax.nn.relu(jnp.dot(h, w1t_ref[...],
                             preferred_element_type=jnp.float32) + b1_ref[...])
    out_ref[...] = jnp.dot(h1, w2t_ref[...],
                           preferred_element_type=jnp.float32) + b2_ref[...]


def _run_mlp(xf, aggf, fwx, fwa, fb, w1t, b1, w2t, b2, blk, interpret=False):
    rows, w = xf.shape
    h = fwx.shape[1]
    h2 = w1t.shape[1]
    grid = (rows // blk,)
    return pl.pallas_call(
        _mlp_body,
        grid=grid,
        in_specs=[
            pl.BlockSpec((blk, w), lambda i: (i, 0)),
            pl.BlockSpec((blk, w), lambda i: (i, 0)),
            pl.BlockSpec((w, h), lambda i: (0, 0)),
            pl.BlockSpec((w, h), lambda i: (0, 0)),
            pl.BlockSpec((1, h), lambda i: (0, 0)),
            pl.BlockSpec((h, h2), lambda i: (0, 0)),
            pl.BlockSpec((1, h2), lambda i: (0, 0)),
            pl.BlockSpec((h2, 1), lambda i: (0, 0)),
            pl.BlockSpec((1, 1), lambda i: (0, 0)),
        ],
        out_specs=pl.BlockSpec((blk, 1), lambda i: (i, 0)),
        out_shape=jax.ShapeDtypeStruct((rows, 1), jnp.float32),
        interpret=interpret,
    )(xf, aggf, fwx, fwa, fb, w1t, b1, w2t, b2)


def _pipeline(x, emb, Wq, Wk, v_w, fc_w, fc_b, w1, b1, w2, b2,
              n, w, d, k, b, blk_topk, blk_edge, chunk, blk_mlp,
              interpret=False):
    ns = 4
    np_ = ((n + ns * blk_topk - 1) // (ns * blk_topk)) * (ns * blk_topk)
    half = np_ // ns
    emb_p = jnp.pad(emb, ((0, np_ - n), (0, 0)))

    a_proj, bm_proj = _run_proj(emb_p, Wq, Wk, interpret=interpret)
    idxs = [_run_topk(emb_p, n, k, blk_topk, hh * half, half,
                      interpret=interpret) for hh in range(ns)]

    xt = jnp.pad(x.transpose(1, 0, 2).reshape(n, b * w), ((0, np_ - n), (0, 0)))
    vt = v_w.reshape(d, 1)

    fwx = fc_w[:, :w].T
    fwa = fc_w[:, w:].T
    w1t = w1.T
    w2t = w2.T

    if interpret:
        idx = jnp.concatenate(idxs, axis=0)
        aggt = _run_edge_tc(idx, a_proj, bm_proj, vt, xt, k, blk_edge, chunk,
                            interpret=True)
        aggs = [aggt[hh * half:(hh + 1) * half] for hh in range(ns)]
    else:
        edge_sc = _make_edge_sc(np_, half, k, d, b * w, nb=8)
        aggs = [
            edge_sc(idxs[hh].reshape(half * k), a_proj,
                    bm_proj[hh * half:(hh + 1) * half], v_w.reshape(d), xt)
            for hh in range(ns)
        ]

    preds = []
    for hh in range(ns):
        xf = xt[hh * half:(hh + 1) * half].reshape(half * b, w)
        aggf = aggs[hh].reshape(half * b, w)
        preds.append(_run_mlp(xf, aggf, fwx, fwa, fc_b[None, :], w1t,
                              b1[None, :], w2t, b2[None, :], blk_mlp,
                              interpret=interpret).reshape(half, b))
    return jnp.concatenate(preds, axis=0)[:n, :].T


def kernel(x, emb, Wq, Wk, v_w, fc_w, fc_b, w1, b1, w2, b2):
    return _pipeline(x, emb, Wq, Wk, v_w, fc_w, fc_b, w1, b1, w2, b2,
                     n=_N, w=_W, d=_D, k=_K, b=_B,
                     blk_topk=512, blk_edge=128, chunk=2048, blk_mlp=512)


# speedup vs baseline: 15.0311x; 1.0366x over previous
import functools

import jax
import jax.numpy as jnp
from jax import lax
from jax.experimental import pallas as pl
from jax.experimental.pallas import tpu as pltpu
from jax.experimental.pallas import tpu_sc as plsc

_N = 10000
_W = 128
_D = 64
_H = 64
_K = 16
_B = 4

_NEG = -3.4e38


def _topk_body(n_real, k, blk, row_base, emb_ref, idx_ref, e_scr):
    pid = pl.program_id(0)

    @pl.when(pid == 0)
    def _():
        e = emb_ref[...]
        nrm = jnp.sqrt(jnp.sum(e * e, axis=1, keepdims=True)) + 1e-8
        e_scr[...] = e / nrm

    e_all = e_scr[...]
    rows = e_scr[pl.ds(row_base + pid * blk, blk), :]
    sim = jax.lax.dot_general(
        rows, e_all, (((1,), (1,)), ((), ())),
        preferred_element_type=jnp.float32)

    np_cols = e_all.shape[0]
    col = jax.lax.broadcasted_iota(jnp.int32, (blk, np_cols), 1)
    rowid = (row_base + pid * blk
             + jax.lax.broadcasted_iota(jnp.int32, (blk, np_cols), 0))
    sim = jnp.where(col == rowid, sim - 1e9, sim)
    sim = jnp.where(col >= n_real, _NEG, sim)

    outs = []
    for _ in range(k):
        idx_t = jnp.argmax(sim, axis=1).astype(jnp.int32)
        outs.append(idx_t)
        sim = jnp.where(col == idx_t[:, None], _NEG, sim)
    idx_ref[...] = jnp.stack(outs, axis=1)


def _run_topk(emb_p, n_real, k, blk, row_base, n_rows, interpret=False):
    np_, d = emb_p.shape
    grid = (n_rows // blk,)
    return pl.pallas_call(
        functools.partial(_topk_body, n_real, k, blk, row_base),
        grid=grid,
        in_specs=[
            pl.BlockSpec((np_, d), lambda i: (0, 0)),
        ],
        out_specs=pl.BlockSpec((blk, k), lambda i: (i, 0)),
        out_shape=jax.ShapeDtypeStruct((n_rows, k), jnp.int32),
        scratch_shapes=[pltpu.VMEM((np_, d), jnp.float32)],
        interpret=interpret,
    )(emb_p)


def _proj_body(emb_ref, wq_ref, wk_ref, a_ref, bm_ref):
    e = emb_ref[...]
    aq = jax.lax.dot_general(
        e, wq_ref[...], (((1,), (1,)), ((), ())),
        preferred_element_type=jnp.float32)
    a_ref[...] = jnp.concatenate([aq, jnp.zeros_like(aq)], axis=1)
    bm_ref[...] = jax.lax.dot_general(
        e, wk_ref[...], (((1,), (1,)), ((), ())),
        preferred_element_type=jnp.float32)


def _run_proj(emb_p, wq, wk, interpret=False):
    np_, d = emb_p.shape
    return pl.pallas_call(
        _proj_body,
        out_shape=[
            jax.ShapeDtypeStruct((np_, 2 * d), jnp.float32),
            jax.ShapeDtypeStruct((np_, d), jnp.float32),
        ],
        interpret=interpret,
    )(emb_p, wq, wk)


def _edge_body(k, blk, chunk, idx_ref, a_ref, bm_ref, vt_ref, xt_ref, agg_ref):
    np_cols = a_ref.shape[0]
    aw = a_ref.shape[1]
    d = bm_ref.shape[1]
    bw = xt_ref.shape[1]
    idxb = idx_ref[...]
    bmb = bm_ref[...]
    vt = vt_ref[...]

    scores = []
    for t in range(k):
        tgt = idxb[:, t][:, None]
        ak = jnp.zeros((blk, aw), dtype=jnp.float32)
        for c in range(0, np_cols, chunk):
            cio = c + jax.lax.broadcasted_iota(jnp.int32, (blk, chunk), 1)
            oh = (cio == tgt).astype(jnp.float32)
            ak = ak + jnp.dot(oh, a_ref[pl.ds(c, chunk), :],
                              preferred_element_type=jnp.float32)
        z = jnp.tanh(ak[:, :d] + bmb)
        scores.append(jnp.dot(z, vt, preferred_element_type=jnp.float32)[:, 0])
    s = jnp.stack(scores, axis=1)

    mx = jnp.max(s, axis=1, keepdims=True)
    e = jnp.exp(s - mx)
    attn = e / (jnp.sum(e, axis=1, keepdims=True) + 1e-8)

    acc = jnp.zeros((blk, bw), dtype=jnp.float32)
    for c in range(0, np_cols, chunk):
        cio = c + jax.lax.broadcasted_iota(jnp.int32, (blk, chunk), 1)
        p = jnp.zeros((blk, chunk), dtype=jnp.float32)
        for t in range(k):
            p = p + jnp.where(cio == idxb[:, t][:, None],
                              attn[:, t][:, None], 0.0)
        acc = acc + jnp.dot(p, xt_ref[pl.ds(c, chunk), :],
                            preferred_element_type=jnp.float32)
    agg_ref[...] = acc


def _run_edge_tc(idx, a, bm, vt, xt, k, blk, chunk, interpret=False):
    np_, aw = a.shape
    d = bm.shape[1]
    bw = xt.shape[1]
    grid = (np_ // blk,)
    return pl.pallas_call(
        functools.partial(_edge_body, k, blk, chunk),
        grid=grid,
        in_specs=[
            pl.BlockSpec((blk, k), lambda i: (i, 0)),
            pl.BlockSpec((np_, aw), lambda i: (0, 0)),
            pl.BlockSpec((blk, d), lambda i: (i, 0)),
            pl.BlockSpec((d, 1), lambda i: (0, 0)),
            pl.BlockSpec((np_, bw), lambda i: (0, 0)),
        ],
        out_specs=pl.BlockSpec((blk, bw), lambda i: (i, 0)),
        out_shape=jax.ShapeDtypeStruct((np_, bw), jnp.float32),
        interpret=interpret,
    )(idx, a, bm, vt, xt)


_SC_NC = 2
_SC_NS = 16
_SC_L = 16


def _tanh_sc(z):
    ez = jnp.exp(z + z)
    return 1.0 - 2.0 / (ez + 1.0)


def _make_edge_sc(np_tab, nrows, k, d, bw, nb):
    nw = _SC_NC * _SC_NS
    npw = nrows // nw
    groups = npw // nb
    cpr = bw // _SC_L
    dch = d // _SC_L
    mesh = plsc.VectorSubcoreMesh(core_axis_name="c", subcore_axis_name="s")

    @functools.partial(
        pl.kernel, mesh=mesh,
        compiler_params=pltpu.CompilerParams(needs_layout_passes=False),
        out_type=jax.ShapeDtypeStruct((nrows, bw), jnp.float32),
        scratch_types=[
            pltpu.VMEM((nb * k,), jnp.int32),
            pltpu.VMEM((nb * k, 2 * d), jnp.float32),
            pltpu.VMEM((nb * k, bw), jnp.float32),
            pltpu.VMEM((nb, d), jnp.float32),
            pltpu.VMEM((nb, bw), jnp.float32),
            pltpu.VMEM((d,), jnp.float32),
            pltpu.SemaphoreType.DMA,
        ],
    )
    def edge_sc(idxf_hbm, a_hbm, bm_hbm, vt_hbm, xt_hbm, out_hbm,
                idx_v, arows_v, xrows_v, bm_v, agg_v, vt_v, sem):
        wid = lax.axis_index("s") * _SC_NC + lax.axis_index("c")
        base = wid * npw
        pltpu.sync_copy(vt_hbm, vt_v)
        lanes = lax.iota(jnp.int32, _SC_L)

        def group_body(g, carry):
            node0 = base + g * nb
            pltpu.sync_copy(idxf_hbm.at[pl.ds(node0 * k, nb * k)], idx_v)
            pltpu.async_copy(a_hbm.at[idx_v], arows_v, sem).wait()
            pltpu.async_copy(xt_hbm.at[idx_v], xrows_v, sem).wait()
            pltpu.sync_copy(bm_hbm.at[pl.ds(node0, nb)], bm_v)

            def node_body(nn, ncarry):
                row0 = nn * k
                bmc = [bm_v[nn, pl.ds(c * _SC_L, _SC_L)] for c in range(dch)]
                vc = [vt_v[pl.ds(c * _SC_L, _SC_L)] for c in range(dch)]
                score = jnp.zeros((_SC_L,), jnp.float32)
                for kk in range(k):
                    part = jnp.zeros((_SC_L,), jnp.float32)
                    for c in range(dch):
                        z = arows_v[row0 + kk,
                                    pl.ds(c * _SC_L, _SC_L)] + bmc[c]
                        part = part + vc[c] * _tanh_sc(z)
                    score = jnp.where(lanes == kk, jnp.sum(part), score)
                m = jnp.max(score)
                e = jnp.exp(score - m)
                attn = e / (jnp.sum(e) + 1e-8)

                accs = [jnp.zeros((_SC_L,), jnp.float32) for _ in range(cpr)]
                for kk in range(k):
                    ak = jnp.sum(jnp.where(lanes == kk, attn, 0.0))
                    for c in range(cpr):
                        accs[c] = accs[c] + ak * xrows_v[
                            row0 + kk, pl.ds(c * _SC_L, _SC_L)]
                for c in range(cpr):
                    agg_v[nn, pl.ds(c * _SC_L, _SC_L)] = accs[c]
                return ncarry

            lax.fori_loop(0, nb, node_body, 0)
            pltpu.sync_copy(agg_v, out_hbm.at[pl.ds(node0, nb)])
            return carry

        lax.fori_loop(0, groups, group_body, 0)

    return edge_sc


def _mlp_body(x_ref, agg_ref, fwx_ref, fwa_ref, fb_ref, w1t_ref, b1_ref,
              w2t_ref, b2_ref, out_ref):
    h = jnp.dot(x_ref[...], fwx_ref[...], preferred_element_type=jnp.float32)
    h = h + jnp.dot(agg_ref[...], fwa_ref[...],
                    preferred_element_type=jnp.float32)
    h = jax.nn.relu(h + fb_ref[...])
    h1 = jax.nn.relu(jnp.dot(h, w1t_ref[...],
                             preferred_element_type=jnp.float32) + b1_ref[...])
    out_ref[...] = jnp.dot(h1, w2t_ref[...],
                           preferred_element_type=jnp.float32) + b2_ref[...]


def _run_mlp(xf, aggf, fwx, fwa, fb, w1t, b1, w2t, b2, blk, interpret=False):
    rows, w = xf.shape
    h = fwx.shape[1]
    h2 = w1t.shape[1]
    grid = (rows // blk,)
    return pl.pallas_call(
        _mlp_body,
        grid=grid,
        in_specs=[
            pl.BlockSpec((blk, w), lambda i: (i, 0)),
            pl.BlockSpec((blk, w), lambda i: (i, 0)),
            pl.BlockSpec((w, h), lambda i: (0, 0)),
            pl.BlockSpec((w, h), lambda i: (0, 0)),
            pl.BlockSpec((1, h), lambda i: (0, 0)),
            pl.BlockSpec((h, h2), lambda i: (0, 0)),
            pl.BlockSpec((1, h2), lambda i: (0, 0)),
            pl.BlockSpec((h2, 1), lambda i: (0, 0)),
            pl.BlockSpec((1, 1), lambda i: (0, 0)),
        ],
        out_specs=pl.BlockSpec((blk, 1), lambda i: (i, 0)),
        out_shape=jax.ShapeDtypeStruct((rows, 1), jnp.float32),
        interpret=interpret,
    )(xf, aggf, fwx, fwa, fb, w1t, b1, w2t, b2)


def _pipeline(x, emb, Wq, Wk, v_w, fc_w, fc_b, w1, b1, w2, b2,
              n, w, d, k, b, blk_topk, blk_edge, chunk, blk_mlp,
              interpret=False):
    ns = 4
    np_ = ((n + ns * blk_topk - 1) // (ns * blk_topk)) * (ns * blk_topk)
    half = np_ // ns
    emb_p = jnp.pad(emb, ((0, np_ - n), (0, 0)))

    a_proj, bm_proj = _run_proj(emb_p, Wq, Wk, interpret=interpret)
    idxs = [_run_topk(emb_p, n, k, blk_topk, hh * half, half,
                      interpret=interpret) for hh in range(ns)]

    xt = jnp.pad(x.transpose(1, 0, 2).reshape(n, b * w), ((0, np_ - n), (0, 0)))
    vt = v_w.reshape(d, 1)

    fwx = fc_w[:, :w].T
    fwa = fc_w[:, w:].T
    w1t = w1.T
    w2t = w2.T

    if interpret:
        idx = jnp.concatenate(idxs, axis=0)
        aggt = _run_edge_tc(idx, a_proj, bm_proj, vt, xt, k, blk_edge, chunk,
                            interpret=True)
        aggs = [aggt[hh * half:(hh + 1) * half] for hh in range(ns)]
    else:
        edge_sc = _make_edge_sc(np_, half, k, d, b * w, nb=8)
        aggs = [
            edge_sc(idxs[hh].reshape(half * k), a_proj,
                    bm_proj[hh * half:(hh + 1) * half], v_w.reshape(d), xt)
            for hh in range(ns)
        ]

    preds = []
    for hh in range(ns):
        xf = xt[hh * half:(hh + 1) * half].reshape(half * b, w)
        aggf = aggs[hh].reshape(half * b, w)
        preds.append(_run_mlp(xf, aggf, fwx, fwa, fc_b[None, :], w1t,
                              b1[None, :], w2t, b2[None, :], blk_mlp,
                              interpret=interpret).reshape(half, b))
    return jnp.concatenate(preds, axis=0)[:n, :].T


def kernel(x, emb, Wq, Wk, v_w, fc_w, fc_b, w1, b1, w2, b2):
    return _pipeline(x, emb, Wq, Wk, v_w, fc_w, fc_b, w1, b1, w2, b2,
                     n=_N, w=_W, d=_D, k=_K, b=_B,
                     blk_topk=640, blk_edge=128, chunk=2048, blk_mlp=512)
